# baseline (speedup 1.0000x reference)
import functools
import jax
import jax.numpy as jnp
from jax import lax
from jax.experimental import pallas as pl
from jax.experimental.pallas import tpu as pltpu
from jax.experimental.pallas import tpu_sc as plsc

B, S, D = 4, 4096, 1024
NC, NS, L = 2, 16, 16
NW = NC * NS
S_PER_W = S // NW
CS = 8
NCHUNK = S_PER_W // CS
NBUF = 3
JV = D // L

B_SC = 1
B_TC = B - B_SC
TS = 512

_mesh = plsc.VectorSubcoreMesh(core_axis_name="c", subcore_axis_name="s")


@functools.partial(
    pl.kernel,
    mesh=_mesh,
    out_type=jax.ShapeDtypeStruct((B_SC, S, D), jnp.float32),
    scratch_types=[
        pltpu.VMEM((NBUF, CS, D), jnp.float32),
        pltpu.VMEM((NBUF, B_SC, CS, D), jnp.float32),
        pltpu.SemaphoreType.DMA((NBUF,)),
        pltpu.SemaphoreType.DMA((NBUF,)),
        pltpu.SemaphoreType.DMA((NBUF,)),
    ],
)
def _sc_posadd(x_hbm, pos_hbm, out_hbm, pv, xv, pin_sem, xin_sem, out_sem):
    wid = lax.axis_index("s") * NC + lax.axis_index("c")
    base = wid * S_PER_W

    def in_copies(c, slot):
        s0 = base + c * CS
        h = [pltpu.make_async_copy(
            pos_hbm.at[pl.ds(s0, CS)], pv.at[slot], pin_sem.at[slot])]
        for b in range(B_SC):
            h.append(pltpu.make_async_copy(
                x_hbm.at[B_TC + b, pl.ds(s0, CS)], xv.at[slot, b],
                xin_sem.at[slot]))
        return h

    def out_copies(c, slot):
        s0 = base + c * CS
        return [pltpu.make_async_copy(
            xv.at[slot, b], out_hbm.at[b, pl.ds(s0, CS)], out_sem.at[slot])
            for b in range(B_SC)]

    def start_in(c, slot):
        for cp in in_copies(c, slot):
            cp.start()

    def wait_in(c, slot):
        for cp in in_copies(c, slot):
            cp.wait()

    def start_out(c, slot):
        for cp in out_copies(c, slot):
            cp.start()

    def wait_out(c, slot):
        for cp in out_copies(c, slot):
            cp.wait()

    def compute(slot):
        def body(k2, carry):
            i = k2 >> 4
            jj = k2 & 15
            for u in range(4):
                off = (jj * 4 + u) * L
                p = pv[slot, i, pl.ds(off, L)]
                for b in range(B_SC):
                    plsc.addupdate(xv.at[slot, b, i, pl.ds(off, L)], p)
            return carry
        lax.fori_loop(0, CS * JV // 4, body, 0)

    def step(c, slot, drain, prefetch):
        nslot = (slot + 1) % NBUF
        if drain:
            wait_out(c - 2, nslot)
        if prefetch:
            start_in(c + 1, nslot)
        wait_in(c, slot)
        compute(slot)
        start_out(c, slot)

    start_in(0, 0)
    step(0, 0, drain=False, prefetch=True)
    step(1, 1, drain=False, prefetch=True)
    step(2, 2, drain=True, prefetch=True)

    def group(g, carry):
        for sl in range(NBUF):
            step(3 + g * NBUF + sl, sl, drain=True, prefetch=True)
        return carry
    lax.fori_loop(0, (NCHUNK - NBUF - 1) // NBUF, group, 0)

    step(NCHUNK - 1, 0, drain=True, prefetch=False)
    wait_out(NCHUNK - 2, 2)
    wait_out(NCHUNK - 1, 0)


def _tc_add(x_ref, emb_ref, out_ref):
    out_ref[...] = x_ref[...] + emb_ref[...][None, :, :]


def kernel(x, pos_table):
    out_sc = _sc_posadd(x, pos_table)
    out_tc = pl.pallas_call(
        _tc_add,
        grid=(S // TS,),
        in_specs=[
            pl.BlockSpec((B_TC, TS, D), lambda i: (0, i, 0)),
            pl.BlockSpec((TS, D), lambda i: (i, 0)),
        ],
        out_specs=pl.BlockSpec((B_TC, TS, D), lambda i: (0, i, 0)),
        out_shape=jax.ShapeDtypeStruct((B_TC, S, D), x.dtype),
    )(x, pos_table)
    return jnp.concatenate([out_tc, out_sc], axis=0)


# speedup vs baseline: 1.5398x; 1.5398x over previous
import functools
import jax
import jax.numpy as jnp
from jax import lax
from jax.experimental import pallas as pl
from jax.experimental.pallas import tpu as pltpu
from jax.experimental.pallas import tpu_sc as plsc

B, S, D = 4, 4096, 1024
NC, NS, L = 2, 16, 16
NW = NC * NS
S_PER_W = S // NW
CS = 8
NCHUNK = S_PER_W // CS
NBUF = 3
JV = D // L

_mesh = plsc.VectorSubcoreMesh(core_axis_name="c", subcore_axis_name="s")


@functools.partial(
    pl.kernel,
    mesh=_mesh,
    out_type=jax.ShapeDtypeStruct((B, S, D), jnp.float32),
    scratch_types=[
        pltpu.VMEM((NBUF, CS, D), jnp.float32),
        pltpu.VMEM((NBUF, B, CS, D), jnp.float32),
        pltpu.SemaphoreType.DMA((NBUF,)),
        pltpu.SemaphoreType.DMA((NBUF,)),
        pltpu.SemaphoreType.DMA((NBUF,)),
    ],
)
def _sc_posadd(x_hbm, pos_hbm, out_hbm, pv, xv, pin_sem, xin_sem, out_sem):
    wid = lax.axis_index("s") * NC + lax.axis_index("c")
    base = wid * S_PER_W

    def in_copies(c, slot):
        s0 = base + c * CS
        h = [pltpu.make_async_copy(
            pos_hbm.at[pl.ds(s0, CS)], pv.at[slot], pin_sem.at[slot])]
        for b in range(B):
            h.append(pltpu.make_async_copy(
                x_hbm.at[b, pl.ds(s0, CS)], xv.at[slot, b], xin_sem.at[slot]))
        return h

    def out_copies(c, slot):
        s0 = base + c * CS
        return [pltpu.make_async_copy(
            xv.at[slot, b], out_hbm.at[b, pl.ds(s0, CS)], out_sem.at[slot])
            for b in range(B)]

    def start_in(c, slot):
        for cp in in_copies(c, slot):
            cp.start()

    def wait_in(c, slot):
        for cp in in_copies(c, slot):
            cp.wait()

    def start_out(c, slot):
        for cp in out_copies(c, slot):
            cp.start()

    def wait_out(c, slot):
        for cp in out_copies(c, slot):
            cp.wait()

    def compute(slot):
        @plsc.parallel_loop(0, CS * JV, unroll=8)
        def body(k):
            i = k >> 6
            off = (k & 63) * L
            p = pv[slot, i, pl.ds(off, L)]
            for b in range(B):
                plsc.addupdate(xv.at[slot, b, i, pl.ds(off, L)], p)

    def step(c, slot, drain, prefetch):
        nslot = (slot + 1) % NBUF
        if drain:
            wait_out(c - 2, nslot)
        if prefetch:
            start_in(c + 1, nslot)
        wait_in(c, slot)
        compute(slot)
        start_out(c, slot)

    start_in(0, 0)
    step(0, 0, drain=False, prefetch=True)
    step(1, 1, drain=False, prefetch=True)
    step(2, 2, drain=True, prefetch=True)

    def group(g, carry):
        for sl in range(NBUF):
            step(3 + g * NBUF + sl, sl, drain=True, prefetch=True)
        return carry
    lax.fori_loop(0, (NCHUNK - NBUF - 1) // NBUF, group, 0)

    step(NCHUNK - 1, 0, drain=True, prefetch=False)
    wait_out(NCHUNK - 2, 2)
    wait_out(NCHUNK - 1, 0)


def kernel(x, pos_table):
    return _sc_posadd(x, pos_table)
